# baseline (speedup 1.0000x reference)
import functools

import jax
import jax.numpy as jnp
from jax import lax
from jax.experimental import pallas as pl
from jax.experimental.pallas import tpu as pltpu
from jax.experimental.pallas import tpu_sc as plsc

N = 10000
H = 128
L = 2
C = 4
G = 64

NPAD = 10240
ZROW = 10200
TRASHROW = NPAD - 1
BLK = 256
CH = 128
NW = 32
RPT = NPAD // 16

_f32 = jnp.float32
_i32 = jnp.int32


@functools.lru_cache(maxsize=None)
def _seg_kernel(cluster, n_chunks_w):
  mesh = plsc.VectorSubcoreMesh(core_axis_name="c", subcore_axis_name="s")

  @functools.partial(
      pl.kernel,
      out_type=jax.ShapeDtypeStruct((2, NPAD, H), _f32),
      mesh=mesh,
      scratch_types=[
          pltpu.VMEM((2, 3, CH), _i32),
          pltpu.VMEM((2, CH), _i32),
          pltpu.VMEM((2, CH), _i32),
          pltpu.VMEM((2, CH, H), _f32),
          pltpu.VMEM_SHARED((NPAD, H), _f32),
          pltpu.SemaphoreType.DMA,
          pltpu.SemaphoreType.DMA,
      ],
  )
  def body(edges_hbm, h_hbm, out_hbm,
           idx2, src_v, dst_v, rows_v, agg_sh, semA, semB):
    core = lax.axis_index("c")
    sub = lax.axis_index("s")
    w = sub * 2 + core
    sems = (semA, semB)

    def _zrow(i, carry):
      for k in range(H // 16):
        rows_v[0, i, pl.ds(k * 16, 16)] = jnp.zeros((16,), _f32)
      return carry
    lax.fori_loop(0, CH, _zrow, 0)
    for u in range(RPT // CH):
      pltpu.sync_copy(rows_v.at[0], agg_sh.at[pl.ds(sub * RPT + u * CH, CH)])
    plsc.subcore_barrier()

    def _stage(p, cid):
      pltpu.sync_copy(edges_hbm.at[cid], idx2.at[p])
      for k in range(CH // 16):
        sl = pl.ds(k * 16, 16)
        dst_v[p, sl] = idx2[p, 1, sl]
        lv = idx2[p, 2, sl]
        sv = idx2[p, 0, sl]
        src_v[p, sl] = jnp.where(lv == cluster, sv, ZROW)
      pltpu.make_async_copy(h_hbm.at[src_v.at[p]], rows_v.at[p],
                            sems[p]).start()

    def _drain(p):
      pltpu.make_async_copy(h_hbm.at[src_v.at[p]], rows_v.at[p],
                            sems[p]).wait()
      pltpu.sync_copy(rows_v.at[p], agg_sh.at[dst_v.at[p]], add=True)

    base = w * n_chunks_w
    _stage(0, base)

    def _pair(j, carry):
      cid = base + 2 * j
      _stage(1, cid + 1)
      _drain(0)
      _stage(0, cid + 2)
      _drain(1)
      return carry
    lax.fori_loop(0, n_chunks_w // 2 - 1, _pair, 0)
    _stage(1, base + n_chunks_w - 1)
    _drain(0)
    _drain(1)
    plsc.subcore_barrier()

    for u in range(RPT // CH):
      r0 = sub * RPT + u * CH
      pltpu.sync_copy(agg_sh.at[pl.ds(r0, CH)], out_hbm.at[core, pl.ds(r0, CH)])

  return body


@functools.lru_cache(maxsize=None)
def _mlp_a(cluster):
  fc = float(cluster)

  def body(h_ref, p0_ref, p1_ref, lab_ref, w1_ref, b1_ref, h1_ref, st_ref):
    pid = pl.program_id(0)

    @pl.when(pid == 0)
    def _():
      st_ref[...] = jnp.zeros_like(st_ref)

    agg = h_ref[...] + p0_ref[...] + p1_ref[...]
    h1 = jnp.dot(agg, w1_ref[...], preferred_element_type=_f32, precision=lax.Precision.HIGHEST) + b1_ref[...]
    mask = lab_ref[...] == fc
    h1m = jnp.where(mask, h1, 0.0)
    s1 = jnp.sum(h1m, axis=0, keepdims=True)
    s2 = jnp.sum(h1m * h1m, axis=0, keepdims=True)
    cnt = jnp.sum(mask.astype(_f32))
    upd = jnp.concatenate(
        [s1, s2, jnp.full((1, H), cnt, _f32), jnp.zeros((5, H), _f32)], axis=0)
    st_ref[...] += upd
    h1_ref[...] = h1

  nb = NPAD // BLK
  return pl.pallas_call(
      body,
      grid=(nb,),
      in_specs=[
          pl.BlockSpec((BLK, H), lambda i: (i, 0)),
          pl.BlockSpec((BLK, H), lambda i: (i, 0)),
          pl.BlockSpec((BLK, H), lambda i: (i, 0)),
          pl.BlockSpec((BLK, 1), lambda i: (i, 0)),
          pl.BlockSpec((H, H), lambda i: (0, 0)),
          pl.BlockSpec((1, H), lambda i: (0, 0)),
      ],
      out_specs=[
          pl.BlockSpec((BLK, H), lambda i: (i, 0)),
          pl.BlockSpec((8, H), lambda i: (0, 0)),
      ],
      out_shape=[
          jax.ShapeDtypeStruct((NPAD, H), _f32),
          jax.ShapeDtypeStruct((8, H), _f32),
      ],
  )


@functools.lru_cache(maxsize=None)
def _mlp_b(cluster):
  fc = float(cluster)

  def body(h1_ref, h_ref, lab_ref, st_ref, g1_ref, be1_ref, w2_ref, b2_ref,
           o_ref):
    s1 = st_ref[0:1, :]
    s2 = st_ref[1:2, :]
    cnt = jnp.maximum(st_ref[2:3, :], 1.0)
    m = s1 / cnt
    v = jnp.maximum(s2 / cnt - m * m, 0.0)
    inv = 1.0 / jnp.sqrt(v + 1e-5)
    h1 = h1_ref[...]
    xn = g1_ref[...] * (h1 - m) * inv + be1_ref[...]
    r = jnp.maximum(xn, 0.0)
    out = jnp.dot(r, w2_ref[...], preferred_element_type=_f32, precision=lax.Precision.HIGHEST) + b2_ref[...]
    mask = lab_ref[...] == fc
    o_ref[...] = jnp.where(mask, out, h_ref[...])

  nb = NPAD // BLK
  return pl.pallas_call(
      body,
      grid=(nb,),
      in_specs=[
          pl.BlockSpec((BLK, H), lambda i: (i, 0)),
          pl.BlockSpec((BLK, H), lambda i: (i, 0)),
          pl.BlockSpec((BLK, 1), lambda i: (i, 0)),
          pl.BlockSpec((8, H), lambda i: (0, 0)),
          pl.BlockSpec((1, H), lambda i: (0, 0)),
          pl.BlockSpec((1, H), lambda i: (0, 0)),
          pl.BlockSpec((H, H), lambda i: (0, 0)),
          pl.BlockSpec((1, H), lambda i: (0, 0)),
      ],
      out_specs=pl.BlockSpec((BLK, H), lambda i: (i, 0)),
      out_shape=jax.ShapeDtypeStruct((NPAD, H), _f32),
  )


def _pool_body(h_ref, b_ref, o_ref):
  pid = pl.program_id(0)

  @pl.when(pid == 0)
  def _():
    o_ref[...] = jnp.zeros_like(o_ref)

  gids = lax.broadcasted_iota(_i32, (G, BLK), 0)
  oh = (gids == b_ref[0]).astype(_f32)
  o_ref[...] += jnp.dot(oh, h_ref[...], preferred_element_type=_f32, precision=lax.Precision.HIGHEST)


_pool_call = pl.pallas_call(
    _pool_body,
    grid=(NPAD // BLK,),
    in_specs=[
        pl.BlockSpec((BLK, H), lambda i: (i, 0)),
        pl.BlockSpec((1, 1, BLK), lambda i: (i, 0, 0)),
    ],
    out_specs=pl.BlockSpec((G, H), lambda i: (0, 0)),
    out_shape=jax.ShapeDtypeStruct((G, H), _f32),
)


def _head_body(p0_ref, p1_ref, wa_ref, wb_ref, bp1_ref, gp_ref, bep_ref,
               wp2_ref, bp2_ref, o_ref):
  h1 = (jnp.dot(p0_ref[...], wa_ref[...], preferred_element_type=_f32, precision=lax.Precision.HIGHEST)
        + jnp.dot(p1_ref[...], wb_ref[...], preferred_element_type=_f32, precision=lax.Precision.HIGHEST)
        + bp1_ref[...])
  m = jnp.mean(h1, axis=0, keepdims=True)
  v = jnp.mean((h1 - m) ** 2, axis=0, keepdims=True)
  xn = gp_ref[...] * (h1 - m) / jnp.sqrt(v + 1e-5) + bep_ref[...]
  r = jnp.maximum(xn, 0.0)
  o_ref[...] = jnp.dot(r, wp2_ref[...], preferred_element_type=_f32, precision=lax.Precision.HIGHEST) + bp2_ref[...]


_head_call = pl.pallas_call(
    _head_body,
    out_shape=jax.ShapeDtypeStruct((G, H), _f32),
)


def kernel(x, edge_index, batch, W1, b1, g1, be1, W2, b2,
           Wp1, bp1, gp, bep, Wp2, bp2):
  E = edge_index.shape[1]
  n_chunks_w = 2 * (-(-E // (NW * CH * 2)))
  EPAD = n_chunks_w * NW * CH

  labf = x[:, 0].astype(_f32)
  lab_pad = jnp.pad(labf[:, None], ((0, NPAD - N), (0, 0)), constant_values=-1.0)
  h = jnp.pad(x[:, 1:].astype(_f32), ((0, NPAD - N), (0, 0)))

  src = jnp.pad(edge_index[0].astype(_i32), (0, EPAD - E), constant_values=ZROW)
  dst = jnp.pad(edge_index[1].astype(_i32), (0, EPAD - E),
                constant_values=TRASHROW)
  labdst = jnp.pad(labf.astype(_i32)[edge_index[1]], (0, EPAD - E),
                   constant_values=-1)
  edges3 = jnp.stack(
      [src.reshape(-1, CH), dst.reshape(-1, CH), labdst.reshape(-1, CH)],
      axis=1)

  batch3 = jnp.pad(batch.astype(_i32), (0, NPAD - N),
                   constant_values=G).reshape(NPAD // BLK, 1, BLK)

  pools = []
  for t in range(L):
    for c in range(C):
      i = t * C + c
      P = _seg_kernel(c, n_chunks_w)(edges3, h)
      h1, st = _mlp_a(c)(h, P[0], P[1], lab_pad, W1[i], b1[i].reshape(1, H))
      h = _mlp_b(c)(h1, h, lab_pad, st, g1[i].reshape(1, H),
                    be1[i].reshape(1, H), W2[i], b2[i].reshape(1, H))
    pools.append(_pool_call(h, batch3))

  return _head_call(pools[0], pools[1], Wp1[:H], Wp1[H:],
                    bp1.reshape(1, H), gp.reshape(1, H), bep.reshape(1, H),
                    Wp2, bp2.reshape(1, H))


# speedup vs baseline: 11.5372x; 11.5372x over previous
import functools

import jax
import jax.numpy as jnp
from jax import lax
from jax.experimental import pallas as pl
from jax.experimental.pallas import tpu as pltpu
from jax.experimental.pallas import tpu_sc as plsc

N = 10000
H = 128
L = 2
C = 4
G = 64

NPAD = 10240
ZROW = 10200
TRASHROW = NPAD - 1
BLK = 256
ECH = 1024
FPT = H // 32

_f32 = jnp.float32
_i32 = jnp.int32


@functools.lru_cache(maxsize=None)
def _seg_kernel(cluster, n_chunks):
  mesh = plsc.VectorSubcoreMesh(core_axis_name="c", subcore_axis_name="s")

  scratch = ([pltpu.VMEM((NPAD,), _f32) for _ in range(FPT)]
             + [pltpu.VMEM((NPAD,), _f32) for _ in range(FPT)]
             + [pltpu.VMEM((2, ECH), _i32),
                pltpu.SemaphoreType.DMA, pltpu.SemaphoreType.DMA])

  @functools.partial(
      pl.kernel,
      out_type=jax.ShapeDtypeStruct((H, NPAD), _f32),
      mesh=mesh,
      scratch_types=scratch,
      compiler_params=pltpu.CompilerParams(needs_layout_passes=False),
  )
  def body(edges_hbm, ht_hbm, out_hbm, *refs):
    hc = refs[:FPT]
    ac = refs[FPT:2 * FPT]
    ev = refs[2 * FPT]
    sems = (refs[2 * FPT + 1], refs[2 * FPT + 2])
    core = lax.axis_index("c")
    sub = lax.axis_index("s")
    tid = sub * 2 + core
    fbase = tid * FPT

    for f in range(FPT):
      pltpu.sync_copy(ht_hbm.at[fbase + f], hc[f])

    def _zero(i, carry):
      for f in range(FPT):
        ac[f][pl.ds(i * 16, 16)] = jnp.zeros((16,), _f32)
      return carry
    lax.fori_loop(0, NPAD // 16, _zero, 0)

    def _start(p, cid):
      pltpu.make_async_copy(edges_hbm.at[pl.ds(cid * ECH, ECH)], ev.at[p],
                            sems[p]).start()

    def _wait(p, cid):
      pltpu.make_async_copy(edges_hbm.at[pl.ds(cid * ECH, ECH)], ev.at[p],
                            sems[p]).wait()

    def _process(p):
      def _vec(k, carry):
        for u in range(4):
          sl = pl.ds(k * 64 + u * 16, 16)
          pk = ev[p, sl]
          srcv = pk & 0x3FFF
          dstv = lax.shift_right_logical(pk, 14) & 0x3FFF
          lv = lax.shift_right_logical(pk, 28)
          srcv = jnp.where(lv == cluster, srcv, ZROW)
          for f in range(FPT):
            vals = plsc.load_gather(hc[f], [srcv])
            plsc.addupdate_scatter(ac[f], [dstv], vals)
        return carry
      lax.fori_loop(0, ECH // 64, _vec, 0)

    npairs = n_chunks // 2
    _start(0, 0)

    def _pair(j, carry):
      c0 = 2 * j
      _start(1, c0 + 1)
      _wait(0, c0)
      _process(0)

      @pl.when(j + 1 < npairs)
      def _():
        _start(0, c0 + 2)
      _wait(1, c0 + 1)
      _process(1)
      return carry
    lax.fori_loop(0, npairs, _pair, 0)

    for f in range(FPT):
      pltpu.sync_copy(ac[f], out_hbm.at[fbase + f])

  return body


@functools.lru_cache(maxsize=None)
def _mlp_a(cluster):
  fc = float(cluster)

  def body(ht_ref, p_ref, lab_ref, w1t_ref, b1_ref, h1_ref, st_ref):
    pid = pl.program_id(0)

    @pl.when(pid == 0)
    def _():
      st_ref[...] = jnp.zeros_like(st_ref)

    agg = ht_ref[...] + p_ref[...]
    h1 = jnp.dot(w1t_ref[...], agg, preferred_element_type=_f32,
                 precision=lax.Precision.HIGHEST) + b1_ref[...]
    mask = lab_ref[...] == fc
    h1m = jnp.where(mask, h1, 0.0)
    s1 = jnp.sum(h1m, axis=1, keepdims=True)
    s2 = jnp.sum(h1m * h1m, axis=1, keepdims=True)
    cnt = jnp.sum(mask.astype(_f32))
    upd = jnp.concatenate(
        [s1, s2, jnp.full((H, 1), cnt, _f32), jnp.zeros((H, 5), _f32)], axis=1)
    st_ref[...] += upd
    h1_ref[...] = h1

  nb = NPAD // BLK
  return pl.pallas_call(
      body,
      grid=(nb,),
      in_specs=[
          pl.BlockSpec((H, BLK), lambda i: (0, i)),
          pl.BlockSpec((H, BLK), lambda i: (0, i)),
          pl.BlockSpec((1, BLK), lambda i: (0, i)),
          pl.BlockSpec((H, H), lambda i: (0, 0)),
          pl.BlockSpec((H, 1), lambda i: (0, 0)),
      ],
      out_specs=[
          pl.BlockSpec((H, BLK), lambda i: (0, i)),
          pl.BlockSpec((H, 8), lambda i: (0, 0)),
      ],
      out_shape=[
          jax.ShapeDtypeStruct((H, NPAD), _f32),
          jax.ShapeDtypeStruct((H, 8), _f32),
      ],
  )


@functools.lru_cache(maxsize=None)
def _mlp_b(cluster):
  fc = float(cluster)

  def body(h1_ref, ht_ref, lab_ref, st_ref, g1_ref, be1_ref, w2t_ref, b2_ref,
           o_ref):
    s1 = st_ref[:, 0:1]
    s2 = st_ref[:, 1:2]
    cnt = jnp.maximum(st_ref[:, 2:3], 1.0)
    m = s1 / cnt
    v = jnp.maximum(s2 / cnt - m * m, 0.0)
    inv = 1.0 / jnp.sqrt(v + 1e-5)
    h1 = h1_ref[...]
    xn = g1_ref[...] * (h1 - m) * inv + be1_ref[...]
    r = jnp.maximum(xn, 0.0)
    out = jnp.dot(w2t_ref[...], r, preferred_element_type=_f32,
                  precision=lax.Precision.HIGHEST) + b2_ref[...]
    mask = lab_ref[...] == fc
    o_ref[...] = jnp.where(mask, out, ht_ref[...])

  nb = NPAD // BLK
  return pl.pallas_call(
      body,
      grid=(nb,),
      in_specs=[
          pl.BlockSpec((H, BLK), lambda i: (0, i)),
          pl.BlockSpec((H, BLK), lambda i: (0, i)),
          pl.BlockSpec((1, BLK), lambda i: (0, i)),
          pl.BlockSpec((H, 8), lambda i: (0, 0)),
          pl.BlockSpec((H, 1), lambda i: (0, 0)),
          pl.BlockSpec((H, 1), lambda i: (0, 0)),
          pl.BlockSpec((H, H), lambda i: (0, 0)),
          pl.BlockSpec((H, 1), lambda i: (0, 0)),
      ],
      out_specs=pl.BlockSpec((H, BLK), lambda i: (0, i)),
      out_shape=jax.ShapeDtypeStruct((H, NPAD), _f32),
  )


def _pool_body(ht_ref, b_ref, o_ref):
  pid = pl.program_id(0)

  @pl.when(pid == 0)
  def _():
    o_ref[...] = jnp.zeros_like(o_ref)

  gids = lax.broadcasted_iota(_i32, (G, BLK), 0)
  oh = (gids == b_ref[...]).astype(_f32)
  o_ref[...] += lax.dot_general(ht_ref[...], oh, (((1,), (1,)), ((), ())),
                                preferred_element_type=_f32,
                                precision=lax.Precision.HIGHEST)


_pool_call = pl.pallas_call(
    _pool_body,
    grid=(NPAD // BLK,),
    in_specs=[
        pl.BlockSpec((H, BLK), lambda i: (0, i)),
        pl.BlockSpec((1, BLK), lambda i: (0, i)),
    ],
    out_specs=pl.BlockSpec((H, G), lambda i: (0, 0)),
    out_shape=jax.ShapeDtypeStruct((H, G), _f32),
)


def _head_body(p0_ref, p1_ref, wa_ref, wb_ref, bp1_ref, gp_ref, bep_ref,
               wp2_ref, bp2_ref, o_ref):
  cdims = (((0,), (0,)), ((), ()))
  h1 = (lax.dot_general(p0_ref[...], wa_ref[...], cdims,
                        preferred_element_type=_f32,
                        precision=lax.Precision.HIGHEST)
        + lax.dot_general(p1_ref[...], wb_ref[...], cdims,
                          preferred_element_type=_f32,
                          precision=lax.Precision.HIGHEST)
        + bp1_ref[...])
  m = jnp.mean(h1, axis=0, keepdims=True)
  v = jnp.mean((h1 - m) ** 2, axis=0, keepdims=True)
  xn = gp_ref[...] * (h1 - m) / jnp.sqrt(v + 1e-5) + bep_ref[...]
  r = jnp.maximum(xn, 0.0)
  o_ref[...] = jnp.dot(r, wp2_ref[...], preferred_element_type=_f32,
                       precision=lax.Precision.HIGHEST) + bp2_ref[...]


_head_call = pl.pallas_call(
    _head_body,
    out_shape=jax.ShapeDtypeStruct((G, H), _f32),
)


def kernel(x, edge_index, batch, W1, b1, g1, be1, W2, b2,
           Wp1, bp1, gp, bep, Wp2, bp2):
  E = edge_index.shape[1]
  n_chunks = 2 * (-(-E // (2 * ECH)))
  EPAD = n_chunks * ECH

  labf = x[:, 0].astype(_f32)
  lab_row = jnp.pad(labf[None, :], ((0, 0), (0, NPAD - N)),
                    constant_values=-1.0)
  hT = jnp.pad(x[:, 1:].astype(_f32).T, ((0, 0), (0, NPAD - N)))

  src = edge_index[0].astype(_i32)
  dst = edge_index[1].astype(_i32)
  lab_dst = labf.astype(_i32)[dst]
  packed = src | (dst << 14) | (lab_dst << 28)
  pad_rec = ZROW | (TRASHROW << 14)
  epack = jnp.pad(packed, (0, EPAD - E), constant_values=pad_rec)

  batch_row = jnp.pad(batch.astype(_i32)[None, :], ((0, 0), (0, NPAD - N)),
                      constant_values=G)

  W1T = jnp.swapaxes(W1, 1, 2)
  W2T = jnp.swapaxes(W2, 1, 2)

  pools = []
  hcur = hT
  for t in range(L):
    for c in range(C):
      i = t * C + c
      P = _seg_kernel(c, n_chunks)(epack, hcur)
      h1, st = _mlp_a(c)(hcur, P, lab_row, W1T[i], b1[i].reshape(H, 1))
      hcur = _mlp_b(c)(h1, hcur, lab_row, st, g1[i].reshape(H, 1),
                       be1[i].reshape(H, 1), W2T[i], b2[i].reshape(H, 1))
    pools.append(_pool_call(hcur, batch_row))

  return _head_call(pools[0], pools[1], Wp1[:H], Wp1[H:],
                    bp1.reshape(1, H), gp.reshape(1, H), bep.reshape(1, H),
                    Wp2, bp2.reshape(1, H))


# speedup vs baseline: 22.0016x; 1.9070x over previous
import functools

import jax
import jax.numpy as jnp
from jax import lax
from jax.experimental import pallas as pl
from jax.experimental.pallas import tpu as pltpu
from jax.experimental.pallas import tpu_sc as plsc

N = 10000
H = 128
L = 2
C = 4
G = 64

NPAD = 10240
ZROW = 10200
TRASHROW = NPAD - 1
BLK = 256
ECH = 1024
FPT = H // 32

_f32 = jnp.float32
_i32 = jnp.int32


_PAD_REC = ZROW | (TRASHROW << 14)


@functools.lru_cache(maxsize=None)
def _count_kernel(epad):
  mesh = plsc.VectorSubcoreMesh(core_axis_name="c", subcore_axis_name="s")
  sl = epad // 32

  @functools.partial(
      pl.kernel,
      out_type=jax.ShapeDtypeStruct((32, C, 16), _i32),
      mesh=mesh,
      scratch_types=[
          pltpu.VMEM((sl,), _i32),
          pltpu.VMEM((C, 16), _i32),
      ],
      compiler_params=pltpu.CompilerParams(needs_layout_passes=False),
  )
  def body(edges_hbm, out_hbm, sl_v, cnt_v):
    core = lax.axis_index("c")
    sub = lax.axis_index("s")
    tid = sub * 2 + core
    pltpu.sync_copy(edges_hbm.at[pl.ds(tid * sl, sl)], sl_v)
    zero = jnp.zeros((16,), _i32)

    def _vec(k, accs):
      pk = sl_v[pl.ds(k * 16, 16)]
      lv = lax.shift_right_logical(pk, 28)
      return tuple(a + jnp.where(lv == c, 1, 0) for c, a in enumerate(accs))
    accs = lax.fori_loop(0, sl // 16, _vec, (zero, zero, zero, zero))
    for c in range(C):
      cnt_v[c, pl.ds(0, 16)] = accs[c]
    pltpu.sync_copy(cnt_v, out_hbm.at[tid])

  return body


@functools.lru_cache(maxsize=None)
def _compact_kernel(epad, e2pad):
  mesh = plsc.VectorSubcoreMesh(core_axis_name="c", subcore_axis_name="s")
  sl = epad // 32

  scratch = ([pltpu.VMEM((sl,), _i32) for _ in range(C)]
             + [pltpu.VMEM((sl,), _i32),
                pltpu.VMEM((C, 16), _i32),
                pltpu.VMEM((C, 2, 16), _i32),
                pltpu.VMEM((16,), _i32)])

  @functools.partial(
      pl.kernel,
      out_type=jax.ShapeDtypeStruct((e2pad,), _i32),
      mesh=mesh,
      scratch_types=scratch,
      compiler_params=pltpu.CompilerParams(needs_layout_passes=False),
  )
  def body(edges_hbm, bases_hbm, meta_hbm, out_hbm, *refs):
    lb = refs[:C]
    sl_v = refs[C]
    tb_v = refs[C + 1]
    meta_v = refs[C + 2]
    inert = refs[C + 3]
    core = lax.axis_index("c")
    sub = lax.axis_index("s")
    tid = sub * 2 + core
    pltpu.sync_copy(edges_hbm.at[pl.ds(tid * sl, sl)], sl_v)
    pltpu.sync_copy(bases_hbm.at[tid], tb_v)
    pltpu.sync_copy(meta_hbm, meta_v)
    inert[pl.ds(0, 16)] = jnp.full((16,), _PAD_REC, _i32)

    def _fill(i, carry):
      for c in range(C):
        lb[c][pl.ds(i * 16, 16)] = jnp.full((16,), _PAD_REC, _i32)
      return carry
    lax.fori_loop(0, sl // 16, _fill, 0)

    zero = jnp.zeros((16,), _i32)

    def _vec(k, locs):
      pk = sl_v[pl.ds(k * 16, 16)]
      lv = lax.shift_right_logical(pk, 28)
      new = []
      for c in range(C):
        m = lv == c
        cs = plsc.cumsum(jnp.where(m, 1, 0))
        pos = locs[c] + cs - 1
        plsc.store_scatter(lb[c], [pos], pk, mask=m)
        new.append(locs[c] + plsc.all_reduce_population_count(m))
      return tuple(new)
    locs = lax.fori_loop(0, sl // 16, _vec, (zero, zero, zero, zero))

    for c in range(C):
      cnt = jnp.max(locs[c])
      tbs16 = jnp.max(tb_v[c, pl.ds(0, 16)]) >> 4
      nch16 = (cnt + 15) >> 4

      def _wb(i, carry):
        pltpu.sync_copy(lb[c].at[pl.ds(i * 16, 16)],
                        out_hbm.at[pl.ds((tbs16 + i) * 16, 16)])
        return carry
      lax.fori_loop(0, nch16, _wb, 0)

      @pl.when(tid == 31)
      def _():
        end16 = (jnp.max(meta_v[c, 1, pl.ds(0, 16)])
                 + jnp.max(meta_v[c, 0, pl.ds(0, 16)])) * (ECH // 16)
        start16 = tbs16 + nch16
        nfill = end16 - start16

        def _fl(i, carry):
          pltpu.sync_copy(inert, out_hbm.at[pl.ds((start16 + i) * 16, 16)])
          return carry
        lax.fori_loop(0, nfill, _fl, 0)

  return body


@functools.lru_cache(maxsize=None)
def _seg_kernel(cluster):
  mesh = plsc.VectorSubcoreMesh(core_axis_name="c", subcore_axis_name="s")

  scratch = ([pltpu.VMEM((NPAD,), _f32) for _ in range(FPT)]
             + [pltpu.VMEM((NPAD,), _f32) for _ in range(FPT)]
             + [pltpu.VMEM((2, ECH), _i32),
                pltpu.VMEM((2, 16), _i32),
                pltpu.SemaphoreType.DMA, pltpu.SemaphoreType.DMA])

  @functools.partial(
      pl.kernel,
      out_type=jax.ShapeDtypeStruct((H, NPAD), _f32),
      mesh=mesh,
      scratch_types=scratch,
      compiler_params=pltpu.CompilerParams(needs_layout_passes=False),
  )
  def body(edges_hbm, ht_hbm, meta_hbm, out_hbm, *refs):
    hc = refs[:FPT]
    ac = refs[FPT:2 * FPT]
    ev = refs[2 * FPT]
    meta_v = refs[2 * FPT + 1]
    sems = (refs[2 * FPT + 2], refs[2 * FPT + 3])
    core = lax.axis_index("c")
    sub = lax.axis_index("s")
    tid = sub * 2 + core
    fbase = tid * FPT

    pltpu.sync_copy(meta_hbm.at[cluster], meta_v)
    for f in range(FPT):
      pltpu.sync_copy(ht_hbm.at[fbase + f], hc[f])

    def _zero(i, carry):
      for f in range(FPT):
        ac[f][pl.ds(i * 16, 16)] = jnp.zeros((16,), _f32)
      return carry
    lax.fori_loop(0, NPAD // 16, _zero, 0)

    nc = jnp.max(meta_v[0, pl.ds(0, 16)])
    cb0 = jnp.max(meta_v[1, pl.ds(0, 16)])
    npairs = nc >> 1

    def _start(p, cid):
      pltpu.make_async_copy(edges_hbm.at[pl.ds((cb0 + cid) * ECH, ECH)],
                            ev.at[p], sems[p]).start()

    def _wait(p, cid):
      pltpu.make_async_copy(edges_hbm.at[pl.ds((cb0 + cid) * ECH, ECH)],
                            ev.at[p], sems[p]).wait()

    def _process(p):
      def _vec(k, carry):
        for u in range(4):
          sl = pl.ds(k * 64 + u * 16, 16)
          pk = ev[p, sl]
          srcv = pk & 0x3FFF
          dstv = lax.shift_right_logical(pk, 14) & 0x3FFF
          lv = lax.shift_right_logical(pk, 28)
          srcv = jnp.where(lv == cluster, srcv, ZROW)
          for f in range(FPT):
            vals = plsc.load_gather(hc[f], [srcv])
            plsc.addupdate_scatter(ac[f], [dstv], vals)
        return carry
      lax.fori_loop(0, ECH // 64, _vec, 0)

    _start(0, 0)

    def _pair(j, carry):
      c0 = 2 * j
      _start(1, c0 + 1)
      _wait(0, c0)
      _process(0)

      @pl.when(j + 1 < npairs)
      def _():
        _start(0, c0 + 2)
      _wait(1, c0 + 1)
      _process(1)
      return carry
    lax.fori_loop(0, npairs, _pair, 0)

    for f in range(FPT):
      pltpu.sync_copy(ac[f], out_hbm.at[fbase + f])

  return body


@functools.lru_cache(maxsize=None)
def _mlp_a(cluster):
  fc = float(cluster)

  def body(ht_ref, p_ref, lab_ref, w1t_ref, b1_ref, h1_ref, st_ref):
    pid = pl.program_id(0)

    @pl.when(pid == 0)
    def _():
      st_ref[...] = jnp.zeros_like(st_ref)

    agg = ht_ref[...] + p_ref[...]
    h1 = jnp.dot(w1t_ref[...], agg, preferred_element_type=_f32,
                 precision=lax.Precision.HIGHEST) + b1_ref[...]
    mask = lab_ref[...] == fc
    h1m = jnp.where(mask, h1, 0.0)
    s1 = jnp.sum(h1m, axis=1, keepdims=True)
    s2 = jnp.sum(h1m * h1m, axis=1, keepdims=True)
    cnt = jnp.sum(mask.astype(_f32))
    upd = jnp.concatenate(
        [s1, s2, jnp.full((H, 1), cnt, _f32), jnp.zeros((H, 5), _f32)], axis=1)
    st_ref[...] += upd
    h1_ref[...] = h1

  nb = NPAD // BLK
  return pl.pallas_call(
      body,
      grid=(nb,),
      in_specs=[
          pl.BlockSpec((H, BLK), lambda i: (0, i)),
          pl.BlockSpec((H, BLK), lambda i: (0, i)),
          pl.BlockSpec((1, BLK), lambda i: (0, i)),
          pl.BlockSpec((H, H), lambda i: (0, 0)),
          pl.BlockSpec((H, 1), lambda i: (0, 0)),
      ],
      out_specs=[
          pl.BlockSpec((H, BLK), lambda i: (0, i)),
          pl.BlockSpec((H, 8), lambda i: (0, 0)),
      ],
      out_shape=[
          jax.ShapeDtypeStruct((H, NPAD), _f32),
          jax.ShapeDtypeStruct((H, 8), _f32),
      ],
  )


@functools.lru_cache(maxsize=None)
def _mlp_b(cluster):
  fc = float(cluster)

  def body(h1_ref, ht_ref, lab_ref, st_ref, g1_ref, be1_ref, w2t_ref, b2_ref,
           o_ref):
    s1 = st_ref[:, 0:1]
    s2 = st_ref[:, 1:2]
    cnt = jnp.maximum(st_ref[:, 2:3], 1.0)
    m = s1 / cnt
    v = jnp.maximum(s2 / cnt - m * m, 0.0)
    inv = 1.0 / jnp.sqrt(v + 1e-5)
    h1 = h1_ref[...]
    xn = g1_ref[...] * (h1 - m) * inv + be1_ref[...]
    r = jnp.maximum(xn, 0.0)
    out = jnp.dot(w2t_ref[...], r, preferred_element_type=_f32,
                  precision=lax.Precision.HIGHEST) + b2_ref[...]
    mask = lab_ref[...] == fc
    o_ref[...] = jnp.where(mask, out, ht_ref[...])

  nb = NPAD // BLK
  return pl.pallas_call(
      body,
      grid=(nb,),
      in_specs=[
          pl.BlockSpec((H, BLK), lambda i: (0, i)),
          pl.BlockSpec((H, BLK), lambda i: (0, i)),
          pl.BlockSpec((1, BLK), lambda i: (0, i)),
          pl.BlockSpec((H, 8), lambda i: (0, 0)),
          pl.BlockSpec((H, 1), lambda i: (0, 0)),
          pl.BlockSpec((H, 1), lambda i: (0, 0)),
          pl.BlockSpec((H, H), lambda i: (0, 0)),
          pl.BlockSpec((H, 1), lambda i: (0, 0)),
      ],
      out_specs=pl.BlockSpec((H, BLK), lambda i: (0, i)),
      out_shape=jax.ShapeDtypeStruct((H, NPAD), _f32),
  )


def _pool_body(ht_ref, b_ref, o_ref):
  pid = pl.program_id(0)

  @pl.when(pid == 0)
  def _():
    o_ref[...] = jnp.zeros_like(o_ref)

  gids = lax.broadcasted_iota(_i32, (G, BLK), 0)
  oh = (gids == b_ref[...]).astype(_f32)
  o_ref[...] += lax.dot_general(ht_ref[...], oh, (((1,), (1,)), ((), ())),
                                preferred_element_type=_f32,
                                precision=lax.Precision.HIGHEST)


_pool_call = pl.pallas_call(
    _pool_body,
    grid=(NPAD // BLK,),
    in_specs=[
        pl.BlockSpec((H, BLK), lambda i: (0, i)),
        pl.BlockSpec((1, BLK), lambda i: (0, i)),
    ],
    out_specs=pl.BlockSpec((H, G), lambda i: (0, 0)),
    out_shape=jax.ShapeDtypeStruct((H, G), _f32),
)


def _head_body(p0_ref, p1_ref, wa_ref, wb_ref, bp1_ref, gp_ref, bep_ref,
               wp2_ref, bp2_ref, o_ref):
  cdims = (((0,), (0,)), ((), ()))
  h1 = (lax.dot_general(p0_ref[...], wa_ref[...], cdims,
                        preferred_element_type=_f32,
                        precision=lax.Precision.HIGHEST)
        + lax.dot_general(p1_ref[...], wb_ref[...], cdims,
                          preferred_element_type=_f32,
                          precision=lax.Precision.HIGHEST)
        + bp1_ref[...])
  m = jnp.mean(h1, axis=0, keepdims=True)
  v = jnp.mean((h1 - m) ** 2, axis=0, keepdims=True)
  xn = gp_ref[...] * (h1 - m) / jnp.sqrt(v + 1e-5) + bep_ref[...]
  r = jnp.maximum(xn, 0.0)
  o_ref[...] = jnp.dot(r, wp2_ref[...], preferred_element_type=_f32,
                       precision=lax.Precision.HIGHEST) + bp2_ref[...]


_head_call = pl.pallas_call(
    _head_body,
    out_shape=jax.ShapeDtypeStruct((G, H), _f32),
)


def kernel(x, edge_index, batch, W1, b1, g1, be1, W2, b2,
           Wp1, bp1, gp, bep, Wp2, bp2):
  E = edge_index.shape[1]
  n_chunks = 2 * (-(-E // (2 * ECH)))
  EPAD = n_chunks * ECH

  labf = x[:, 0].astype(_f32)
  lab_row = jnp.pad(labf[None, :], ((0, 0), (0, NPAD - N)),
                    constant_values=-1.0)
  hT = jnp.pad(x[:, 1:].astype(_f32).T, ((0, 0), (0, NPAD - N)))

  src = edge_index[0].astype(_i32)
  dst = edge_index[1].astype(_i32)
  lab_dst = labf.astype(_i32)[dst]
  packed = src | (dst << 14) | (lab_dst << 28)
  epack = jnp.pad(packed, (0, EPAD - E), constant_values=_PAD_REC)

  counts = _count_kernel(EPAD)(epack)
  cnts = counts.sum(-1)
  cnt16 = ((cnts + 15) // 16) * 16
  tsum = cnt16.sum(0)
  sz = jnp.maximum(((tsum + 2 * ECH - 1) // (2 * ECH)) * (2 * ECH), 2 * ECH)
  cb = jnp.concatenate([jnp.zeros((1,), _i32),
                        jnp.cumsum(sz)[:-1].astype(_i32)])
  tb = cb[None, :] + (jnp.cumsum(cnt16, axis=0) - cnt16)
  bases_splat = jnp.broadcast_to(tb[:, :, None], (32, C, 16)).astype(_i32)
  meta = jnp.broadcast_to(
      jnp.stack([sz // ECH, cb // ECH], axis=1)[:, :, None],
      (C, 2, 16)).astype(_i32)
  E2PAD = EPAD + 16384
  ecomp = _compact_kernel(EPAD, E2PAD)(epack, bases_splat, meta)

  batch_row = jnp.pad(batch.astype(_i32)[None, :], ((0, 0), (0, NPAD - N)),
                      constant_values=G)

  W1T = jnp.swapaxes(W1, 1, 2)
  W2T = jnp.swapaxes(W2, 1, 2)

  pools = []
  hcur = hT
  for t in range(L):
    for c in range(C):
      i = t * C + c
      P = _seg_kernel(c)(ecomp, hcur, meta)
      h1, st = _mlp_a(c)(hcur, P, lab_row, W1T[i], b1[i].reshape(H, 1))
      hcur = _mlp_b(c)(h1, hcur, lab_row, st, g1[i].reshape(H, 1),
                       be1[i].reshape(H, 1), W2T[i], b2[i].reshape(H, 1))
    pools.append(_pool_call(hcur, batch_row))

  return _head_call(pools[0], pools[1], Wp1[:H], Wp1[H:],
                    bp1.reshape(1, H), gp.reshape(1, H), bep.reshape(1, H),
                    Wp2, bp2.reshape(1, H))


# speedup vs baseline: 41.2130x; 1.8732x over previous
import functools

import jax
import jax.numpy as jnp
from jax import lax
from jax.experimental import pallas as pl
from jax.experimental.pallas import tpu as pltpu
from jax.experimental.pallas import tpu_sc as plsc

N = 10000
H = 128
L = 2
C = 4
G = 64

NPAD = 10240
ZROW = 10200
TRASHROW = NPAD - 1
BLK = 256
ECH = 1024
FPT = H // 32

_f32 = jnp.float32
_i32 = jnp.int32


_PAD_REC = ZROW | (TRASHROW << 14)


@functools.lru_cache(maxsize=None)
def _count_kernel(epad):
  mesh = plsc.VectorSubcoreMesh(core_axis_name="c", subcore_axis_name="s")
  sl = epad // 32

  @functools.partial(
      pl.kernel,
      out_type=jax.ShapeDtypeStruct((32, C, 16), _i32),
      mesh=mesh,
      scratch_types=[
          pltpu.VMEM((sl,), _i32),
          pltpu.VMEM((NPAD,), _i32),
          pltpu.VMEM((C, 16), _i32),
      ],
      compiler_params=pltpu.CompilerParams(needs_layout_passes=False),
  )
  def body(edges_hbm, lab_hbm, out_hbm, sl_v, lab_v, cnt_v):
    core = lax.axis_index("c")
    sub = lax.axis_index("s")
    tid = sub * 2 + core
    pltpu.sync_copy(edges_hbm.at[pl.ds(tid * sl, sl)], sl_v)
    pltpu.sync_copy(lab_hbm, lab_v)
    zero = jnp.zeros((16,), _i32)

    def _vec(k, accs):
      pk = sl_v[pl.ds(k * 16, 16)]
      dstv = lax.shift_right_logical(pk, 14) & 0x3FFF
      lv = plsc.load_gather(lab_v, [dstv])
      return tuple(a + jnp.where(lv == c, 1, 0) for c, a in enumerate(accs))
    accs = lax.fori_loop(0, sl // 16, _vec, (zero, zero, zero, zero))
    for c in range(C):
      cnt_v[c, pl.ds(0, 16)] = accs[c]
    pltpu.sync_copy(cnt_v, out_hbm.at[tid])

  return body


@functools.lru_cache(maxsize=None)
def _compact_kernel(epad, e2pad):
  mesh = plsc.VectorSubcoreMesh(core_axis_name="c", subcore_axis_name="s")
  sl = epad // 32

  scratch = ([pltpu.VMEM((sl,), _i32) for _ in range(C)]
             + [pltpu.VMEM((sl,), _i32),
                pltpu.VMEM((NPAD,), _i32),
                pltpu.VMEM((C, 16), _i32),
                pltpu.VMEM((C, 2, 16), _i32),
                pltpu.VMEM((16,), _i32)])

  @functools.partial(
      pl.kernel,
      out_type=jax.ShapeDtypeStruct((e2pad,), _i32),
      mesh=mesh,
      scratch_types=scratch,
      compiler_params=pltpu.CompilerParams(needs_layout_passes=False),
  )
  def body(edges_hbm, lab_hbm, bases_hbm, meta_hbm, out_hbm, *refs):
    lb = refs[:C]
    sl_v = refs[C]
    lab_v = refs[C + 1]
    tb_v = refs[C + 2]
    meta_v = refs[C + 3]
    inert = refs[C + 4]
    core = lax.axis_index("c")
    sub = lax.axis_index("s")
    tid = sub * 2 + core
    pltpu.sync_copy(edges_hbm.at[pl.ds(tid * sl, sl)], sl_v)
    pltpu.sync_copy(lab_hbm, lab_v)
    pltpu.sync_copy(bases_hbm.at[tid], tb_v)
    pltpu.sync_copy(meta_hbm, meta_v)
    inert[pl.ds(0, 16)] = jnp.full((16,), _PAD_REC, _i32)

    def _fill(i, carry):
      for c in range(C):
        lb[c][pl.ds(i * 16, 16)] = jnp.full((16,), _PAD_REC, _i32)
      return carry
    lax.fori_loop(0, sl // 16, _fill, 0)

    zero = jnp.zeros((16,), _i32)

    def _vec(k, locs):
      pk = sl_v[pl.ds(k * 16, 16)]
      dstv = lax.shift_right_logical(pk, 14) & 0x3FFF
      lv = plsc.load_gather(lab_v, [dstv])
      new = []
      for c in range(C):
        m = lv == c
        cs = plsc.cumsum(jnp.where(m, 1, 0))
        pos = locs[c] + cs - 1
        plsc.store_scatter(lb[c], [pos], pk | (c << 28), mask=m)
        new.append(locs[c] + plsc.all_reduce_population_count(m))
      return tuple(new)
    locs = lax.fori_loop(0, sl // 16, _vec, (zero, zero, zero, zero))

    for c in range(C):
      cnt = jnp.max(locs[c])
      tbs16 = jnp.max(tb_v[c, pl.ds(0, 16)]) >> 4
      nch16 = (cnt + 15) >> 4

      def _wb(i, carry):
        pltpu.sync_copy(lb[c].at[pl.ds(i * 16, 16)],
                        out_hbm.at[pl.ds((tbs16 + i) * 16, 16)])
        return carry
      lax.fori_loop(0, nch16, _wb, 0)

      @pl.when(tid == 31)
      def _():
        end16 = (jnp.max(meta_v[c, 1, pl.ds(0, 16)])
                 + jnp.max(meta_v[c, 0, pl.ds(0, 16)])) * (ECH // 16)
        start16 = tbs16 + nch16
        nfill = end16 - start16

        def _fl(i, carry):
          pltpu.sync_copy(inert, out_hbm.at[pl.ds((start16 + i) * 16, 16)])
          return carry
        lax.fori_loop(0, nfill, _fl, 0)

  return body


@functools.lru_cache(maxsize=None)
def _seg_kernel(cluster):
  mesh = plsc.VectorSubcoreMesh(core_axis_name="c", subcore_axis_name="s")

  scratch = ([pltpu.VMEM((NPAD,), _f32) for _ in range(FPT)]
             + [pltpu.VMEM((NPAD,), _f32) for _ in range(FPT)]
             + [pltpu.VMEM((2, ECH), _i32),
                pltpu.VMEM((2, 16), _i32),
                pltpu.SemaphoreType.DMA, pltpu.SemaphoreType.DMA])

  @functools.partial(
      pl.kernel,
      out_type=jax.ShapeDtypeStruct((H, NPAD), _f32),
      mesh=mesh,
      scratch_types=scratch,
      compiler_params=pltpu.CompilerParams(needs_layout_passes=False),
  )
  def body(edges_hbm, ht_hbm, meta_hbm, out_hbm, *refs):
    hc = refs[:FPT]
    ac = refs[FPT:2 * FPT]
    ev = refs[2 * FPT]
    meta_v = refs[2 * FPT + 1]
    sems = (refs[2 * FPT + 2], refs[2 * FPT + 3])
    core = lax.axis_index("c")
    sub = lax.axis_index("s")
    tid = sub * 2 + core
    fbase = tid * FPT

    pltpu.sync_copy(meta_hbm.at[cluster], meta_v)
    for f in range(FPT):
      pltpu.sync_copy(ht_hbm.at[fbase + f], hc[f])

    def _zero(i, carry):
      for f in range(FPT):
        ac[f][pl.ds(i * 16, 16)] = jnp.zeros((16,), _f32)
      return carry
    lax.fori_loop(0, NPAD // 16, _zero, 0)

    nc = jnp.max(meta_v[0, pl.ds(0, 16)])
    cb0 = jnp.max(meta_v[1, pl.ds(0, 16)])
    npairs = nc >> 1

    def _start(p, cid):
      pltpu.make_async_copy(edges_hbm.at[pl.ds((cb0 + cid) * ECH, ECH)],
                            ev.at[p], sems[p]).start()

    def _wait(p, cid):
      pltpu.make_async_copy(edges_hbm.at[pl.ds((cb0 + cid) * ECH, ECH)],
                            ev.at[p], sems[p]).wait()

    def _process(p):
      def _vec(k, carry):
        for u in range(4):
          sl = pl.ds(k * 64 + u * 16, 16)
          pk = ev[p, sl]
          srcv = pk & 0x3FFF
          dstv = lax.shift_right_logical(pk, 14) & 0x3FFF
          lv = lax.shift_right_logical(pk, 28)
          srcv = jnp.where(lv == cluster, srcv, ZROW)
          for f in range(FPT):
            vals = plsc.load_gather(hc[f], [srcv])
            plsc.addupdate_scatter(ac[f], [dstv], vals)
        return carry
      lax.fori_loop(0, ECH // 64, _vec, 0)

    _start(0, 0)

    def _pair(j, carry):
      c0 = 2 * j
      _start(1, c0 + 1)
      _wait(0, c0)
      _process(0)

      @pl.when(j + 1 < npairs)
      def _():
        _start(0, c0 + 2)
      _wait(1, c0 + 1)
      _process(1)
      return carry
    lax.fori_loop(0, npairs, _pair, 0)

    for f in range(FPT):
      pltpu.sync_copy(ac[f], out_hbm.at[fbase + f])

  return body


@functools.lru_cache(maxsize=None)
def _mlp_fused(cluster):
  fc = float(cluster)

  def body(ht_ref, p_ref, lab_ref, w1t_ref, b1_ref, g1_ref, be1_ref,
           w2t_ref, b2_ref, o_ref, h1s, st):
    ph = pl.program_id(0)
    i = pl.program_id(1)
    mask = lab_ref[...] == fc

    @pl.when(ph == 0)
    def _():
      @pl.when(i == 0)
      def _():
        st[...] = jnp.zeros_like(st)
      agg = ht_ref[...] + p_ref[...]
      h1 = jnp.dot(w1t_ref[...], agg, preferred_element_type=_f32,
                   precision=lax.Precision.HIGHEST) + b1_ref[...]
      h1m = jnp.where(mask, h1, 0.0)
      s1 = jnp.sum(h1m, axis=1, keepdims=True)
      s2 = jnp.sum(h1m * h1m, axis=1, keepdims=True)
      cnt = jnp.sum(mask.astype(_f32))
      st[...] += jnp.concatenate(
          [s1, s2, jnp.full((H, 1), cnt, _f32), jnp.zeros((H, 5), _f32)],
          axis=1)
      h1s[:, pl.ds(i * BLK, BLK)] = h1

    @pl.when(ph == 1)
    def _():
      s1 = st[:, 0:1]
      s2 = st[:, 1:2]
      cnt = jnp.maximum(st[:, 2:3], 1.0)
      m = s1 / cnt
      v = jnp.maximum(s2 / cnt - m * m, 0.0)
      inv = 1.0 / jnp.sqrt(v + 1e-5)
      h1 = h1s[:, pl.ds(i * BLK, BLK)]
      xn = g1_ref[...] * (h1 - m) * inv + be1_ref[...]
      r = jnp.maximum(xn, 0.0)
      out = jnp.dot(w2t_ref[...], r, preferred_element_type=_f32,
                    precision=lax.Precision.HIGHEST) + b2_ref[...]
      o_ref[...] = jnp.where(mask, out, ht_ref[...])

  nb = NPAD // BLK
  return pl.pallas_call(
      body,
      grid=(2, nb),
      in_specs=[
          pl.BlockSpec((H, BLK), lambda p, i: (0, i)),
          pl.BlockSpec((H, BLK), lambda p, i: (0, i)),
          pl.BlockSpec((1, BLK), lambda p, i: (0, i)),
          pl.BlockSpec((H, H), lambda p, i: (0, 0)),
          pl.BlockSpec((H, 1), lambda p, i: (0, 0)),
          pl.BlockSpec((H, 1), lambda p, i: (0, 0)),
          pl.BlockSpec((H, 1), lambda p, i: (0, 0)),
          pl.BlockSpec((H, H), lambda p, i: (0, 0)),
          pl.BlockSpec((H, 1), lambda p, i: (0, 0)),
      ],
      out_specs=pl.BlockSpec((H, BLK), lambda p, i: (0, i)),
      out_shape=jax.ShapeDtypeStruct((H, NPAD), _f32),
      scratch_shapes=[
          pltpu.VMEM((H, NPAD), _f32),
          pltpu.VMEM((H, 8), _f32),
      ],
  )


def _pool_body(ht_ref, b_ref, o_ref):
  pid = pl.program_id(0)

  @pl.when(pid == 0)
  def _():
    o_ref[...] = jnp.zeros_like(o_ref)

  gids = lax.broadcasted_iota(_i32, (G, BLK), 0)
  oh = (gids == b_ref[...]).astype(_f32)
  o_ref[...] += lax.dot_general(ht_ref[...], oh, (((1,), (1,)), ((), ())),
                                preferred_element_type=_f32,
                                precision=lax.Precision.HIGHEST)


_pool_call = pl.pallas_call(
    _pool_body,
    grid=(NPAD // BLK,),
    in_specs=[
        pl.BlockSpec((H, BLK), lambda i: (0, i)),
        pl.BlockSpec((1, BLK), lambda i: (0, i)),
    ],
    out_specs=pl.BlockSpec((H, G), lambda i: (0, 0)),
    out_shape=jax.ShapeDtypeStruct((H, G), _f32),
)


def _head_body(p0_ref, p1_ref, wa_ref, wb_ref, bp1_ref, gp_ref, bep_ref,
               wp2_ref, bp2_ref, o_ref):
  cdims = (((0,), (0,)), ((), ()))
  h1 = (lax.dot_general(p0_ref[...], wa_ref[...], cdims,
                        preferred_element_type=_f32,
                        precision=lax.Precision.HIGHEST)
        + lax.dot_general(p1_ref[...], wb_ref[...], cdims,
                          preferred_element_type=_f32,
                          precision=lax.Precision.HIGHEST)
        + bp1_ref[...])
  m = jnp.mean(h1, axis=0, keepdims=True)
  v = jnp.mean((h1 - m) ** 2, axis=0, keepdims=True)
  xn = gp_ref[...] * (h1 - m) / jnp.sqrt(v + 1e-5) + bep_ref[...]
  r = jnp.maximum(xn, 0.0)
  o_ref[...] = jnp.dot(r, wp2_ref[...], preferred_element_type=_f32,
                       precision=lax.Precision.HIGHEST) + bp2_ref[...]


_head_call = pl.pallas_call(
    _head_body,
    out_shape=jax.ShapeDtypeStruct((G, H), _f32),
)


def kernel(x, edge_index, batch, W1, b1, g1, be1, W2, b2,
           Wp1, bp1, gp, bep, Wp2, bp2):
  E = edge_index.shape[1]
  n_chunks = 2 * (-(-E // (2 * ECH)))
  EPAD = n_chunks * ECH

  labf = x[:, 0].astype(_f32)
  lab_row = jnp.pad(labf[None, :], ((0, 0), (0, NPAD - N)),
                    constant_values=-1.0)
  hT = jnp.pad(x[:, 1:].astype(_f32).T, ((0, 0), (0, NPAD - N)))

  src = edge_index[0].astype(_i32)
  dst = edge_index[1].astype(_i32)
  packed = src | (dst << 14)
  epack = jnp.pad(packed, (0, EPAD - E), constant_values=_PAD_REC)
  lab_i = jnp.pad(labf.astype(_i32), (0, NPAD - N), constant_values=-1)

  counts = _count_kernel(EPAD)(epack, lab_i)
  cnts = counts.sum(-1)
  cnt16 = ((cnts + 15) // 16) * 16
  tsum = cnt16.sum(0)
  sz = jnp.maximum(((tsum + 2 * ECH - 1) // (2 * ECH)) * (2 * ECH), 2 * ECH)
  cb = jnp.concatenate([jnp.zeros((1,), _i32),
                        jnp.cumsum(sz)[:-1].astype(_i32)])
  tb = cb[None, :] + (jnp.cumsum(cnt16, axis=0) - cnt16)
  bases_splat = jnp.broadcast_to(tb[:, :, None], (32, C, 16)).astype(_i32)
  meta = jnp.broadcast_to(
      jnp.stack([sz // ECH, cb // ECH], axis=1)[:, :, None],
      (C, 2, 16)).astype(_i32)
  E2PAD = EPAD + 16384
  ecomp = _compact_kernel(EPAD, E2PAD)(epack, lab_i, bases_splat, meta)

  batch_row = jnp.pad(batch.astype(_i32)[None, :], ((0, 0), (0, NPAD - N)),
                      constant_values=G)

  W1T = jnp.swapaxes(W1, 1, 2)
  W2T = jnp.swapaxes(W2, 1, 2)

  pools = []
  hcur = hT
  for t in range(L):
    for c in range(C):
      i = t * C + c
      P = _seg_kernel(c)(ecomp, hcur, meta)
      hcur = _mlp_fused(c)(hcur, P, lab_row, W1T[i], b1[i].reshape(H, 1),
                           g1[i].reshape(H, 1), be1[i].reshape(H, 1),
                           W2T[i], b2[i].reshape(H, 1))
    pools.append(_pool_call(hcur, batch_row))

  return _head_call(pools[0], pools[1], Wp1[:H], Wp1[H:],
                    bp1.reshape(1, H), gp.reshape(1, H), bep.reshape(1, H),
                    Wp2, bp2.reshape(1, H))
